# baseline (speedup 1.0000x reference)
import functools

import jax
import jax.numpy as jnp
from jax import lax
from jax.experimental import pallas as pl
from jax.experimental.pallas import tpu as pltpu
from jax.experimental.pallas import tpu_sc as plsc

D = 128
NC, NS = 2, 16
NW = NC * NS
CHUNK = 128
NBUF = 4
SPB = 32


def _make_gather(B):
    assert B % (NW * CHUNK) == 0
    b_per_w = B // NW
    n_slots = b_per_w // CHUNK
    assert n_slots % SPB == 0 and n_slots % NBUF == 0
    n_quads = n_slots // NBUF
    n_blocks = n_slots // SPB
    mesh = plsc.VectorSubcoreMesh(core_axis_name="c", subcore_axis_name="s")

    @functools.partial(
        pl.kernel,
        out_type=jax.ShapeDtypeStruct((B, D), jnp.float32),
        mesh=mesh,
        scratch_types=[
            pltpu.VMEM((2, SPB, CHUNK), jnp.int32),
            pltpu.VMEM((NBUF, CHUNK, D), jnp.float32),
            pltpu.SemaphoreType.DMA((NBUF,)),
            pltpu.SemaphoreType.DMA((NBUF,)),
            pltpu.SemaphoreType.DMA,
        ],
    )
    def gather_kernel(idx_hbm, table_hbm, out_hbm, idx_blk, rows_v,
                      sem_g, sem_w, sem_i):
        wid = lax.axis_index("s") * NC + lax.axis_index("c")
        row_base = wid * n_slots

        def blk_copy(kb):
            return pltpu.make_async_copy(
                idx_hbm.at[pl.ds(row_base + kb * SPB, SPB)],
                idx_blk.at[lax.rem(kb, 2)], sem_i)

        def gath_copy(s, b):
            return pltpu.make_async_copy(
                table_hbm.at[idx_blk.at[lax.rem(s // SPB, 2), lax.rem(s, SPB)]],
                rows_v.at[b], sem_g.at[b])

        def out_copy(s, b):
            return pltpu.make_async_copy(
                rows_v.at[b],
                out_hbm.at[pl.ds((row_base + s) * CHUNK, CHUNK)],
                sem_w.at[b])

        pltpu.sync_copy(idx_hbm.at[pl.ds(row_base, SPB)], idx_blk.at[0])
        blk_copy(1).start()
        for b in range(2):
            gath_copy(b, b).start()

        def body(q, carry):
            for b4 in range(NBUF):
                s = NBUF * q + b4
                b = b4

                @pl.when(s >= 2)
                def _():
                    out_copy(s - 2, (b + 2) % NBUF).wait()

                sf = s + 2

                @pl.when(sf < n_slots)
                def _():
                    @pl.when(lax.rem(sf, SPB) == 0)
                    def _():
                        blk_copy(sf // SPB).wait()

                    gath_copy(sf, (b + 2) % NBUF).start()

                gath_copy(s, b).wait()
                out_copy(s, b).start()

                @pl.when((lax.rem(s, SPB) == SPB - 1)
                         & (s // SPB + 2 < n_blocks))
                def _():
                    blk_copy(s // SPB + 2).start()
            return carry

        lax.fori_loop(0, n_quads, body, 0)
        out_copy(n_slots - 2, (n_slots - 2) % NBUF).wait()
        out_copy(n_slots - 1, (n_slots - 1) % NBUF).wait()

    return gather_kernel


def kernel(idx, table):
    B_rows, H = idx.shape
    B = B_rows * H
    idx2d = idx.reshape(B // CHUNK, CHUNK)
    out = _make_gather(B)(idx2d, table)
    return out.reshape(B_rows, H, D)


# speedup vs baseline: 1.0224x; 1.0224x over previous
import functools

import jax
import jax.numpy as jnp
from jax import lax
from jax.experimental import pallas as pl
from jax.experimental.pallas import tpu as pltpu
from jax.experimental.pallas import tpu_sc as plsc

D = 128
NC, NS = 2, 16
NW = NC * NS
NBUF = 4
SPB = 32
SPLITS = ((0, 128), (128, 72))


def _make_gather(B_rows, H):
    assert sum(n for _, n in SPLITS) == H
    n_slots = B_rows // NW
    assert B_rows % NW == 0
    assert n_slots % SPB == 0 and n_slots % NBUF == 0
    n_quads = n_slots // NBUF
    n_blocks = n_slots // SPB
    mesh = plsc.VectorSubcoreMesh(core_axis_name="c", subcore_axis_name="s")

    @functools.partial(
        pl.kernel,
        out_type=jax.ShapeDtypeStruct((B_rows, H, D), jnp.float32),
        mesh=mesh,
        scratch_types=[
            pltpu.VMEM((2, SPB, H), jnp.int32),
            pltpu.VMEM((NBUF, H, D), jnp.float32),
            pltpu.SemaphoreType.DMA((NBUF,)),
            pltpu.SemaphoreType.DMA((NBUF,)),
            pltpu.SemaphoreType.DMA,
        ],
    )
    def gather_kernel(idx_hbm, table_hbm, out_hbm, idx_blk, rows_v,
                      sem_g, sem_w, sem_i):
        wid = lax.axis_index("s") * NC + lax.axis_index("c")
        row_base = wid * n_slots

        def blk_copy(kb):
            return pltpu.make_async_copy(
                idx_hbm.at[pl.ds(row_base + kb * SPB, SPB)],
                idx_blk.at[lax.rem(kb, 2)], sem_i)

        def gath_copies(s, b):
            sel = lax.rem(s // SPB, 2)
            r = lax.rem(s, SPB)
            return [
                pltpu.make_async_copy(
                    table_hbm.at[idx_blk.at[sel, r, pl.ds(off, n)]],
                    rows_v.at[b, pl.ds(off, n)], sem_g.at[b])
                for off, n in SPLITS
            ]

        def out_copy(s, b):
            return pltpu.make_async_copy(
                rows_v.at[b], out_hbm.at[row_base + s], sem_w.at[b])

        pltpu.sync_copy(idx_hbm.at[pl.ds(row_base, SPB)], idx_blk.at[0])
        blk_copy(1).start()
        for b in range(2):
            for c in gath_copies(b, b):
                c.start()

        def body(q, carry):
            for b4 in range(NBUF):
                s = NBUF * q + b4
                b = b4

                @pl.when(s >= 2)
                def _():
                    out_copy(s - 2, (b + 2) % NBUF).wait()

                sf = s + 2

                @pl.when(sf < n_slots)
                def _():
                    @pl.when(lax.rem(sf, SPB) == 0)
                    def _():
                        blk_copy(sf // SPB).wait()

                    for c in gath_copies(sf, (b + 2) % NBUF):
                        c.start()

                for c in gath_copies(s, b):
                    c.wait()
                out_copy(s, b).start()

                @pl.when((lax.rem(s, SPB) == SPB - 1)
                         & (s // SPB + 2 < n_blocks))
                def _():
                    blk_copy(s // SPB + 2).start()
            return carry

        lax.fori_loop(0, n_quads, body, 0)
        out_copy(n_slots - 2, (n_slots - 2) % NBUF).wait()
        out_copy(n_slots - 1, (n_slots - 1) % NBUF).wait()

    return gather_kernel


def kernel(idx, table):
    B_rows, H = idx.shape
    return _make_gather(B_rows, H)(idx, table)
